# baseline (speedup 1.0000x reference)
import jax
import jax.numpy as jnp
from jax import lax
from jax.experimental import pallas as pl
from jax.experimental.pallas import tpu as pltpu
from jax.experimental.pallas import tpu_sc as plsc

_BATCH = 16384
_D = 32
_NC = 2
_NS = 16
_L = 16
_NW = _NC * _NS
_BPW = _BATCH // _NW
_CHUNK = 128
_NCHUNK = _BPW // _CHUNK
_GROUPS = _BPW // _L
_GPC = _CHUNK // _L


def _body(uids, iids, utab, itab, ubtab, ibtab, gb, out,
          idx_u, idx_i, hi_u, hi_i, urows, irows, ubblk, ibblk,
          gbv, outv, sem):
    w = lax.axis_index("s") * _NC + lax.axis_index("c")
    pltpu.sync_copy(uids.at[w], idx_u)
    pltpu.sync_copy(iids.at[w], idx_i)
    pltpu.sync_copy(gb, gbv)

    for j in range(_NCHUNK):
        for o in range(0, _CHUNK, _L):
            sl = pl.ds(o, _L)
            hi_u[j, sl] = lax.shift_right_logical(idx_u[j, sl], 4)
            hi_i[j, sl] = lax.shift_right_logical(idx_i[j, sl], 4)

    copies = []
    for j in range(_NCHUNK):
        sl = pl.ds(j * _CHUNK, _CHUNK)
        copies.append(pltpu.async_copy(utab.at[idx_u.at[j]], urows.at[sl], sem))
        copies.append(pltpu.async_copy(itab.at[idx_i.at[j]], irows.at[sl], sem))
        copies.append(pltpu.async_copy(ubtab.at[hi_u.at[j]], ubblk.at[sl], sem))
        copies.append(pltpu.async_copy(ibtab.at[hi_i.at[j]], ibblk.at[sl], sem))
    for c in copies:
        c.wait()

    gvec = gbv[...]

    def group(g, carry):
        j = g // _GPC
        o = (g % _GPC) * _L
        row = lax.broadcasted_iota(jnp.int32, (_L,), 0) + g * _L
        acc = jnp.zeros((_L,), jnp.float32)
        for d in range(_D):
            col = jnp.full((_L,), d, jnp.int32)
            uu = plsc.load_gather(urows, [row, col])
            ii = plsc.load_gather(irows, [row, col])
            acc = acc + uu * ii
        lane_u = jnp.bitwise_and(idx_u[j, pl.ds(o, _L)], 15)
        lane_i = jnp.bitwise_and(idx_i[j, pl.ds(o, _L)], 15)
        bu = plsc.load_gather(ubblk, [row, lane_u])
        bi = plsc.load_gather(ibblk, [row, lane_i])
        t = acc + bu + bi + gvec
        outv[pl.ds(g * _L, _L)] = 1.0 / (1.0 + jnp.exp(-t))
        return carry

    lax.fori_loop(0, _GROUPS, group, 0)
    pltpu.sync_copy(outv, out.at[w])


def kernel(user_ids, item_ids, user_table, item_table,
           user_bias_table, item_bias_table, global_bias):
    uids = user_ids.reshape(_NW, _NCHUNK, _CHUNK)
    iids = item_ids.reshape(_NW, _NCHUNK, _CHUNK)
    ubt = user_bias_table.reshape(-1, _L)
    ibt = item_bias_table.reshape(-1, _L)
    gb16 = jnp.broadcast_to(global_bias.astype(jnp.float32), (_L,))
    mesh = plsc.VectorSubcoreMesh(core_axis_name="c", subcore_axis_name="s")
    k = pl.kernel(
        _body,
        mesh=mesh,
        compiler_params=pltpu.CompilerParams(
            needs_layout_passes=False, use_tc_tiling_on_sc=False),
        out_type=jax.ShapeDtypeStruct((_NW, _BPW), jnp.float32),
        scratch_types=[
            pltpu.VMEM((_NCHUNK, _CHUNK), jnp.int32),
            pltpu.VMEM((_NCHUNK, _CHUNK), jnp.int32),
            pltpu.VMEM((_NCHUNK, _CHUNK), jnp.int32),
            pltpu.VMEM((_NCHUNK, _CHUNK), jnp.int32),
            pltpu.VMEM((_BPW, _D), jnp.float32),
            pltpu.VMEM((_BPW, _D), jnp.float32),
            pltpu.VMEM((_BPW, _L), jnp.float32),
            pltpu.VMEM((_BPW, _L), jnp.float32),
            pltpu.VMEM((_L,), jnp.float32),
            pltpu.VMEM((_BPW,), jnp.float32),
            pltpu.SemaphoreType.DMA,
        ],
    )
    out = k(uids, iids, user_table, item_table, ubt, ibt, gb16)
    return out.reshape(_BATCH, 1)


# speedup vs baseline: 1.0004x; 1.0004x over previous
import jax
import jax.numpy as jnp
from jax import lax
from jax.experimental import pallas as pl
from jax.experimental.pallas import tpu as pltpu
from jax.experimental.pallas import tpu_sc as plsc

_BATCH = 16384
_D = 32
_NC = 2
_NS = 16
_L = 16
_NW = _NC * _NS
_BPW = _BATCH // _NW
_CHUNK = 128
_NCHUNK = _BPW // _CHUNK
_GROUPS = _BPW // _L
_GPC = _CHUNK // _L


def _body(uids, iids, utab, itab, ubtab, ibtab, gb, out,
          idx_u, idx_i, hi_u, hi_i, urows, irows, ubv, ibv,
          gbv, outv, sem):
    w = lax.axis_index("s") * _NC + lax.axis_index("c")
    pltpu.sync_copy(uids.at[w], idx_u)
    pltpu.sync_copy(iids.at[w], idx_i)
    pltpu.sync_copy(gb, gbv)

    for j in range(_NCHUNK):
        for o in range(0, _CHUNK, _L):
            sl = pl.ds(o, _L)
            hi_u[j, sl] = lax.shift_right_logical(idx_u[j, sl], 4)
            hi_i[j, sl] = lax.shift_right_logical(idx_i[j, sl], 4)

    copies = []
    for j in range(_NCHUNK):
        sl = pl.ds(j * _CHUNK, _CHUNK)
        copies.append(pltpu.async_copy(utab.at[idx_u.at[j]], urows.at[sl], sem))
        copies.append(pltpu.async_copy(itab.at[idx_i.at[j]], irows.at[sl], sem))
        copies.append(pltpu.async_copy(ubtab.at[hi_u.at[j]], ubv.at[sl], sem))
        copies.append(pltpu.async_copy(ibtab.at[hi_i.at[j]], ibv.at[sl], sem))
    for c in copies:
        c.wait()

    gvec = gbv[...]

    def group(g, carry):
        j = g // _GPC
        o = (g % _GPC) * _L
        row = lax.broadcasted_iota(jnp.int32, (_L,), 0) + g * _L
        acc = jnp.zeros((_L,), jnp.float32)
        for d in range(_D):
            col = jnp.full((_L,), d, jnp.int32)
            uu = plsc.load_gather(urows, [row, col])
            ii = plsc.load_gather(irows, [row, col])
            acc = acc + uu * ii
        lane_u = jnp.bitwise_and(idx_u[j, pl.ds(o, _L)], 15)
        lane_i = jnp.bitwise_and(idx_i[j, pl.ds(o, _L)], 15)
        bu = plsc.load_gather(ubv, [row, lane_u])
        bi = plsc.load_gather(ibv, [row, lane_i])
        t = acc + bu + bi + gvec
        outv[pl.ds(g * _L, _L)] = 1.0 / (1.0 + jnp.exp(-t))
        return carry

    lax.fori_loop(0, _GROUPS, group, 0)
    pltpu.sync_copy(outv, out.at[w])


def kernel(user_ids, item_ids, user_table, item_table,
           user_bias_table, item_bias_table, global_bias):
    uids = user_ids.reshape(_NW, _NCHUNK, _CHUNK)
    iids = item_ids.reshape(_NW, _NCHUNK, _CHUNK)
    ubt = user_bias_table.reshape(-1, _L)
    ibt = item_bias_table.reshape(-1, _L)
    gb16 = jnp.broadcast_to(global_bias.astype(jnp.float32), (_L,))
    mesh = plsc.VectorSubcoreMesh(core_axis_name="c", subcore_axis_name="s")
    k = pl.kernel(
        _body,
        mesh=mesh,
        compiler_params=pltpu.CompilerParams(
            needs_layout_passes=False, use_tc_tiling_on_sc=False),
        out_type=jax.ShapeDtypeStruct((_NW, _BPW), jnp.float32),
        scratch_types=[
            pltpu.VMEM((_NCHUNK, _CHUNK), jnp.int32),
            pltpu.VMEM((_NCHUNK, _CHUNK), jnp.int32),
            pltpu.VMEM((_NCHUNK, _CHUNK), jnp.int32),
            pltpu.VMEM((_NCHUNK, _CHUNK), jnp.int32),
            pltpu.VMEM((_BPW, _D), jnp.float32),
            pltpu.VMEM((_BPW, _D), jnp.float32),
            pltpu.VMEM((_BPW, _L), jnp.float32),
            pltpu.VMEM((_BPW, _L), jnp.float32),
            pltpu.VMEM((_L,), jnp.float32),
            pltpu.VMEM((_BPW,), jnp.float32),
            pltpu.SemaphoreType.DMA,
        ],
    )
    out = k(uids, iids, user_table, item_table, ubt, ibt, gb16)
    return out.reshape(_BATCH, 1)


# speedup vs baseline: 1.5712x; 1.5706x over previous
import jax
import jax.numpy as jnp
from jax import lax
from jax.experimental import pallas as pl
from jax.experimental.pallas import tpu as pltpu
from jax.experimental.pallas import tpu_sc as plsc

_BATCH = 16384
_D = 32
_NC = 2
_NS = 16
_L = 16
_NW = _NC * _NS
_BPW = _BATCH // _NW
_CHUNK = 128
_NCHUNK = _BPW // _CHUNK
_BGROUPS = _BPW // _L
_GPC = _CHUNK // _L
_GID = 8
_DGROUPS = _BPW // _GID
_BPGRAN = 4 * 8 * _L * 4
_CLS = ((0, 0, 16), (1, 1, 32), (2, 3, 64), (4, 7, 128))


def _wid():
    return lax.axis_index("s") * _NC + lax.axis_index("c")


def _bias_body(uids, iids, ubt, ibt, gb, out,
               idx_u, idx_i, hi_u, hi_i, ubblk, ibblk, gbv, outv, sem):
    w = _wid()
    pltpu.sync_copy(uids.at[w], idx_u)
    pltpu.sync_copy(iids.at[w], idx_i)
    pltpu.sync_copy(gb, gbv)

    for j in range(_NCHUNK):
        for o in range(0, _CHUNK, _L):
            sl = pl.ds(o, _L)
            hi_u[j, sl] = lax.shift_right_logical(idx_u[j, sl], 4)
            hi_i[j, sl] = lax.shift_right_logical(idx_i[j, sl], 4)

    copies = []
    for j in range(_NCHUNK):
        sl = pl.ds(j * _CHUNK, _CHUNK)
        copies.append(pltpu.async_copy(ubt.at[hi_u.at[j]], ubblk.at[sl], sem))
        copies.append(pltpu.async_copy(ibt.at[hi_i.at[j]], ibblk.at[sl], sem))
    for c in copies:
        c.wait()

    gvec = gbv[...]

    def group(g, carry):
        j = g // _GPC
        o = (g % _GPC) * _L
        row = lax.broadcasted_iota(jnp.int32, (_L,), 0) + g * _L
        lane_u = jnp.bitwise_and(idx_u[j, pl.ds(o, _L)], 15)
        lane_i = jnp.bitwise_and(idx_i[j, pl.ds(o, _L)], 15)
        bu = plsc.load_gather(ubblk, [row, lane_u])
        bi = plsc.load_gather(ibblk, [row, lane_i])
        outv[pl.ds(g * _L, _L)] = bu + bi + gvec
        return carry

    lax.fori_loop(0, _BGROUPS, group, 0)
    pltpu.sync_copy(outv, out.at[w])


def _dot_body(uids, iids, ut3, it3, bsum, out,
              idsu, idsi, bsv, outv, bufu, bufi, sema, semb):
    w = _wid()
    base = w * _BPW
    pltpu.sync_copy(uids.at[pl.ds(base, _BPW)], idsu.at[pl.ds(0, _BPW)])
    pltpu.sync_copy(iids.at[pl.ds(base, _BPW)], idsi.at[pl.ds(0, _BPW)])
    pltpu.sync_copy(bsum.at[pl.ds(base, _BPW)], bsv.at[pl.ds(0, _BPW)])

    lane = lax.broadcasted_iota(jnp.int32, (_L,), 0)
    low8 = lane < _GID
    slot = jnp.minimum(lane, _GID - 1)

    def fire(chunk, tab, buf, sem):
        phs = []
        for i in range(_GID):
            sel = jnp.where(lane == i, chunk, 0)
            x = lax.reduce_sum(sel, axes=(0,))
            ph = jnp.bitwise_and(lax.shift_right_logical(x, 4), 7)
            phs.append(ph)
            xt = pl.multiple_of(jnp.bitwise_and(x, -128), 128)
            for lo, hi, nl in _CLS:
                @pl.when(jnp.logical_and(ph >= lo, ph <= hi))
                def _(xt=xt, nl=nl, i=i, tab=tab, buf=buf):
                    pltpu.async_copy(
                        tab.at[:, :, pl.ds(xt, nl)],
                        buf.at[i, :, :, pl.ds(0, nl)], sem)
        return phs

    def drain(phs, tab, buf, sem):
        for i in range(_GID):
            for lo, hi, nl in _CLS:
                @pl.when(jnp.logical_and(phs[i] >= lo, phs[i] <= hi))
                def _(nl=nl, i=i, tab=tab, buf=buf):
                    pltpu.make_async_copy(
                        tab.at[:, :, pl.ds(0, nl)],
                        buf.at[i, :, :, pl.ds(0, nl)], sem).wait()

    def group(g, carry):
        cu = idsu[pl.ds(g * _GID, _L)]
        ci = idsi[pl.ds(g * _GID, _L)]
        phu = fire(cu, ut3, bufu, sema)
        phi = fire(ci, it3, bufi, semb)
        drain(phu, ut3, bufu, sema)
        drain(phi, it3, bufi, semb)

        a3u = jnp.bitwise_and(cu, 127)
        a3i = jnp.bitwise_and(ci, 127)
        acc = jnp.zeros((_L,), jnp.float32)
        for d in range(_D):
            a1 = jnp.full((_L,), d // 8, jnp.int32)
            a2 = jnp.full((_L,), d % 8, jnp.int32)
            uu = plsc.load_gather(bufu, [slot, a1, a2, a3u])
            ii = plsc.load_gather(bufi, [slot, a1, a2, a3i])
            acc = acc + uu * ii
        t = acc + bsv[pl.ds(g * _GID, _L)]
        res = 1.0 / (1.0 + jnp.exp(-t))
        plsc.store_compressed(outv.at[pl.ds(g * _GID, _L)], res, mask=low8)
        return carry

    lax.fori_loop(0, _DGROUPS, group, 0)
    pltpu.sync_copy(outv.at[pl.ds(0, _BPW)], out.at[pl.ds(base, _BPW)])


def kernel(user_ids, item_ids, user_table, item_table,
           user_bias_table, item_bias_table, global_bias):
    mesh = plsc.VectorSubcoreMesh(core_axis_name="c", subcore_axis_name="s")

    uids3 = user_ids.reshape(_NW, _NCHUNK, _CHUNK)
    iids3 = item_ids.reshape(_NW, _NCHUNK, _CHUNK)
    ubt = user_bias_table.reshape(-1, _L)
    ibt = item_bias_table.reshape(-1, _L)
    gb16 = jnp.broadcast_to(global_bias.astype(jnp.float32), (_L,))

    bias_k = pl.kernel(
        _bias_body,
        mesh=mesh,
        compiler_params=pltpu.CompilerParams(
            needs_layout_passes=False, use_tc_tiling_on_sc=False),
        out_type=jax.ShapeDtypeStruct((_NW, _BPW), jnp.float32),
        scratch_types=[
            pltpu.VMEM((_NCHUNK, _CHUNK), jnp.int32),
            pltpu.VMEM((_NCHUNK, _CHUNK), jnp.int32),
            pltpu.VMEM((_NCHUNK, _CHUNK), jnp.int32),
            pltpu.VMEM((_NCHUNK, _CHUNK), jnp.int32),
            pltpu.VMEM((_BPW, _L), jnp.float32),
            pltpu.VMEM((_BPW, _L), jnp.float32),
            pltpu.VMEM((_L,), jnp.float32),
            pltpu.VMEM((_BPW,), jnp.float32),
            pltpu.SemaphoreType.DMA,
        ],
    )
    bsum = bias_k(uids3, iids3, ubt, ibt, gb16).reshape(_BATCH)

    nu = user_table.shape[0]
    ni = item_table.shape[0]
    ut3 = user_table.T.reshape(4, 8, nu)
    it3 = item_table.T.reshape(4, 8, ni)

    dot_k = pl.kernel(
        _dot_body,
        mesh=mesh,
        compiler_params=pltpu.CompilerParams(
            needs_layout_passes=False, use_tc_tiling_on_sc=True),
        out_type=jax.ShapeDtypeStruct((_BATCH,), jnp.float32),
        scratch_types=[
            pltpu.VMEM((_BPW + _L,), jnp.int32),
            pltpu.VMEM((_BPW + _L,), jnp.int32),
            pltpu.VMEM((_BPW + _L,), jnp.float32),
            pltpu.VMEM((_BPW + _L,), jnp.float32),
            pltpu.VMEM((_GID, 4, 8, 128), jnp.float32),
            pltpu.VMEM((_GID, 4, 8, 128), jnp.float32),
            pltpu.SemaphoreType.DMA,
            pltpu.SemaphoreType.DMA,
        ],
    )
    out = dot_k(user_ids, item_ids, ut3, it3, bsum)
    return out.reshape(_BATCH, 1)


# speedup vs baseline: 1.6787x; 1.0684x over previous
import jax
import jax.numpy as jnp
from jax import lax
from jax.experimental import pallas as pl
from jax.experimental.pallas import tpu as pltpu
from jax.experimental.pallas import tpu_sc as plsc

_BATCH = 16384
_D = 32
_NC = 2
_NS = 16
_L = 16
_NW = _NC * _NS
_BPW = _BATCH // _NW
_CHUNK = 128
_NCHUNK = _BPW // _CHUNK
_BGROUPS = _BPW // _L
_GPC = _CHUNK // _L
_GID = 4
_DGROUPS = _BPW // _GID
_BPGRAN = 4 * 8 * _L * 4
_CLS = ((0, 0, 16), (1, 1, 32), (2, 3, 64), (4, 7, 128))


def _wid():
    return lax.axis_index("s") * _NC + lax.axis_index("c")


def _bias_body(uids, iids, ubt, ibt, gb, out,
               idx_u, idx_i, hi_u, hi_i, ubblk, ibblk, gbv, outv, sem):
    w = _wid()
    pltpu.sync_copy(uids.at[w], idx_u)
    pltpu.sync_copy(iids.at[w], idx_i)
    pltpu.sync_copy(gb, gbv)

    for j in range(_NCHUNK):
        for o in range(0, _CHUNK, _L):
            sl = pl.ds(o, _L)
            hi_u[j, sl] = lax.shift_right_logical(idx_u[j, sl], 4)
            hi_i[j, sl] = lax.shift_right_logical(idx_i[j, sl], 4)

    copies = []
    for j in range(_NCHUNK):
        sl = pl.ds(j * _CHUNK, _CHUNK)
        copies.append(pltpu.async_copy(ubt.at[hi_u.at[j]], ubblk.at[sl], sem))
        copies.append(pltpu.async_copy(ibt.at[hi_i.at[j]], ibblk.at[sl], sem))
    for c in copies:
        c.wait()

    gvec = gbv[...]

    def group(g, carry):
        j = g // _GPC
        o = (g % _GPC) * _L
        row = lax.broadcasted_iota(jnp.int32, (_L,), 0) + g * _L
        lane_u = jnp.bitwise_and(idx_u[j, pl.ds(o, _L)], 15)
        lane_i = jnp.bitwise_and(idx_i[j, pl.ds(o, _L)], 15)
        bu = plsc.load_gather(ubblk, [row, lane_u])
        bi = plsc.load_gather(ibblk, [row, lane_i])
        outv[pl.ds(g * _L, _L)] = bu + bi + gvec
        return carry

    lax.fori_loop(0, _BGROUPS, group, 0)
    pltpu.sync_copy(outv, out.at[w])


def _dot_body(uids, iids, ut3, it3, bsum, out,
              idsu, idsi, bsv, outv, bufua, bufia, bufub, bufib,
              sema, semb, semc, semd):
    w = _wid()
    base = w * _BPW
    pltpu.sync_copy(uids.at[pl.ds(base, _BPW)], idsu.at[pl.ds(0, _BPW)])
    pltpu.sync_copy(iids.at[pl.ds(base, _BPW)], idsi.at[pl.ds(0, _BPW)])
    pltpu.sync_copy(bsum.at[pl.ds(base, _BPW)], bsv.at[pl.ds(0, _BPW)])

    lane = lax.broadcasted_iota(jnp.int32, (_L,), 0)
    low8 = lane < _GID
    slot = jnp.minimum(lane, _GID - 1)

    def fire(chunk, tab, buf, sem):
        phs = []
        for i in range(_GID):
            sel = jnp.where(lane == i, chunk, 0)
            x = lax.reduce_sum(sel, axes=(0,))
            ph = jnp.bitwise_and(lax.shift_right_logical(x, 4), 7)
            phs.append(ph)
            xt = pl.multiple_of(jnp.bitwise_and(x, -128), 128)
            for lo, hi, nl in _CLS:
                @pl.when(jnp.logical_and(ph >= lo, ph <= hi))
                def _(xt=xt, nl=nl, i=i, tab=tab, buf=buf):
                    pltpu.async_copy(
                        tab.at[:, :, pl.ds(xt, nl)],
                        buf.at[i, :, :, pl.ds(0, nl)], sem)
        return phs

    def drain(phs, tab, buf, sem):
        for i in range(_GID):
            for lo, hi, nl in _CLS:
                @pl.when(jnp.logical_and(phs[i] >= lo, phs[i] <= hi))
                def _(nl=nl, i=i, tab=tab, buf=buf):
                    pltpu.make_async_copy(
                        tab.at[:, :, pl.ds(0, nl)],
                        buf.at[i, :, :, pl.ds(0, nl)], sem).wait()

    def fire_g(g, bu, bi, su, si):
        cu = idsu[pl.ds(g * _GID, _L)]
        ci = idsi[pl.ds(g * _GID, _L)]
        fire(cu, ut3, bu, su)
        fire(ci, it3, bi, si)

    def finish_g(g, bu, bi, su, si):
        cu = idsu[pl.ds(g * _GID, _L)]
        ci = idsi[pl.ds(g * _GID, _L)]
        phu = [jnp.bitwise_and(lax.shift_right_logical(lax.reduce_sum(
            jnp.where(lane == i, cu, 0), axes=(0,)), 4), 7)
            for i in range(_GID)]
        phi = [jnp.bitwise_and(lax.shift_right_logical(lax.reduce_sum(
            jnp.where(lane == i, ci, 0), axes=(0,)), 4), 7)
            for i in range(_GID)]
        drain(phu, ut3, bu, su)
        drain(phi, it3, bi, si)
        a3u = jnp.bitwise_and(cu, 127)
        a3i = jnp.bitwise_and(ci, 127)
        acc = jnp.zeros((_L,), jnp.float32)
        for d in range(_D):
            a1 = jnp.full((_L,), d // 8, jnp.int32)
            a2 = jnp.full((_L,), d % 8, jnp.int32)
            uu = plsc.load_gather(bu, [slot, a1, a2, a3u])
            ii = plsc.load_gather(bi, [slot, a1, a2, a3i])
            acc = acc + uu * ii
        t = acc + bsv[pl.ds(g * _GID, _L)]
        res = 1.0 / (1.0 + jnp.exp(-t))
        plsc.store_compressed(outv.at[pl.ds(g * _GID, _L)], res, mask=low8)

    fire_g(0, bufua, bufia, sema, semb)

    def pair(p, carry):
        ga = 2 * p
        gb_ = 2 * p + 1
        fire_g(gb_, bufub, bufib, semc, semd)
        finish_g(ga, bufua, bufia, sema, semb)
        fire_g(jnp.bitwise_and(gb_ + 1, _DGROUPS - 1),
               bufua, bufia, sema, semb)
        finish_g(gb_, bufub, bufib, semc, semd)
        return carry

    lax.fori_loop(0, _DGROUPS // 2, pair, 0)
    finish_g(0, bufua, bufia, sema, semb)
    pltpu.sync_copy(outv.at[pl.ds(0, _BPW)], out.at[pl.ds(base, _BPW)])


def kernel(user_ids, item_ids, user_table, item_table,
           user_bias_table, item_bias_table, global_bias):
    mesh = plsc.VectorSubcoreMesh(core_axis_name="c", subcore_axis_name="s")

    uids3 = user_ids.reshape(_NW, _NCHUNK, _CHUNK)
    iids3 = item_ids.reshape(_NW, _NCHUNK, _CHUNK)
    ubt = user_bias_table.reshape(-1, _L)
    ibt = item_bias_table.reshape(-1, _L)
    gb16 = jnp.broadcast_to(global_bias.astype(jnp.float32), (_L,))

    bias_k = pl.kernel(
        _bias_body,
        mesh=mesh,
        compiler_params=pltpu.CompilerParams(
            needs_layout_passes=False, use_tc_tiling_on_sc=False),
        out_type=jax.ShapeDtypeStruct((_NW, _BPW), jnp.float32),
        scratch_types=[
            pltpu.VMEM((_NCHUNK, _CHUNK), jnp.int32),
            pltpu.VMEM((_NCHUNK, _CHUNK), jnp.int32),
            pltpu.VMEM((_NCHUNK, _CHUNK), jnp.int32),
            pltpu.VMEM((_NCHUNK, _CHUNK), jnp.int32),
            pltpu.VMEM((_BPW, _L), jnp.float32),
            pltpu.VMEM((_BPW, _L), jnp.float32),
            pltpu.VMEM((_L,), jnp.float32),
            pltpu.VMEM((_BPW,), jnp.float32),
            pltpu.SemaphoreType.DMA,
        ],
    )
    bsum = bias_k(uids3, iids3, ubt, ibt, gb16).reshape(_BATCH)

    nu = user_table.shape[0]
    ni = item_table.shape[0]
    ut3 = user_table.T.reshape(4, 8, nu)
    it3 = item_table.T.reshape(4, 8, ni)

    dot_k = pl.kernel(
        _dot_body,
        mesh=mesh,
        compiler_params=pltpu.CompilerParams(
            needs_layout_passes=False, use_tc_tiling_on_sc=True),
        out_type=jax.ShapeDtypeStruct((_BATCH,), jnp.float32),
        scratch_types=[
            pltpu.VMEM((_BPW + _L,), jnp.int32),
            pltpu.VMEM((_BPW + _L,), jnp.int32),
            pltpu.VMEM((_BPW + _L,), jnp.float32),
            pltpu.VMEM((_BPW + _L,), jnp.float32),
            pltpu.VMEM((_GID, 4, 8, 128), jnp.float32),
            pltpu.VMEM((_GID, 4, 8, 128), jnp.float32),
            pltpu.VMEM((_GID, 4, 8, 128), jnp.float32),
            pltpu.VMEM((_GID, 4, 8, 128), jnp.float32),
            pltpu.SemaphoreType.DMA,
            pltpu.SemaphoreType.DMA,
            pltpu.SemaphoreType.DMA,
            pltpu.SemaphoreType.DMA,
        ],
    )
    out = dot_k(user_ids, item_ids, ut3, it3, bsum)
    return out.reshape(_BATCH, 1)
